# Initial kernel scaffold; baseline (speedup 1.0000x reference)
import jax
import jax.numpy as jnp
from jax.experimental import pallas as pl


def kernel(x, edge_index, batch, atom_emb, W1, b1, g1, bt1, W2, b2, g2, bt2, Wl, bl):
    raise NotImplementedError("write your pallas kernel here")


# speedup vs baseline: 2.4619x; 2.4619x over previous
import functools

import jax
import jax.numpy as jnp
from jax import lax
from jax.experimental import pallas as pl
from jax.experimental.pallas import tpu as pltpu

N = 10000
H = 128
HH = 64
NF = 9
NG = 64
NPAD = 12288


def _mask_rows(nrows):
    ri = lax.broadcasted_iota(jnp.int32, (nrows, 1), 0)
    return (ri < N).astype(jnp.float32)


def _tc1_body(h0a_ref, h0b_ref, deg2_ref, w1_ref, outa_ref, outb_ref):
    h0 = jnp.concatenate([h0a_ref[...], h0b_ref[...]], axis=1)
    deg = deg2_ref[:, 0:1] + deg2_ref[:, 1:2] + 1.0
    dinv = lax.rsqrt(deg)
    hw = jnp.dot(h0, w1_ref[...], preferred_element_type=jnp.float32)
    hwp = hw * dinv
    outa_ref[...] = hwp[:, :HH]
    outb_ref[...] = hwp[:, HH:]


def _tc2_body(agga_ref, aggb_ref, hwa_ref, hwb_ref, deg2_ref, b_ref, g_ref,
              bt_ref, w2_ref, outa_ref, outb_ref):
    agg = jnp.concatenate([agga_ref[...] + hwa_ref[...],
                           aggb_ref[...] + hwb_ref[...]], axis=1)
    deg = deg2_ref[:, 0:1] + deg2_ref[:, 1:2] + 1.0
    dinv = lax.rsqrt(deg)
    z = agg * dinv + b_ref[...]
    m = _mask_rows(z.shape[0])
    mu = jnp.sum(z * m, axis=0, keepdims=True) / N
    d = (z - mu) * m
    var = jnp.sum(d * d, axis=0, keepdims=True) / N
    h = jnp.maximum((z - mu) * lax.rsqrt(var + 1e-5) * g_ref[...] + bt_ref[...],
                    0.0) * m
    hw = jnp.dot(h, w2_ref[...], preferred_element_type=jnp.float32)
    hwp = hw * dinv
    outa_ref[...] = hwp[:, :HH]
    outb_ref[...] = hwp[:, HH:]


def _tc3_body(agga_ref, aggb_ref, hwa_ref, hwb_ref, deg2_ref, b_ref, g_ref,
              bt_ref, batch_ref, wl_ref, bl_ref, out_ref):
    agg = jnp.concatenate([agga_ref[...] + hwa_ref[...],
                           aggb_ref[...] + hwb_ref[...]], axis=1)
    deg = deg2_ref[:, 0:1] + deg2_ref[:, 1:2] + 1.0
    dinv = lax.rsqrt(deg)
    z = agg * dinv + b_ref[...]
    m = _mask_rows(z.shape[0])
    mu = jnp.sum(z * m, axis=0, keepdims=True) / N
    d = (z - mu) * m
    var = jnp.sum(d * d, axis=0, keepdims=True) / N
    h = jnp.maximum((z - mu) * lax.rsqrt(var + 1e-5) * g_ref[...] + bt_ref[...],
                    0.0)
    hn = h[:N, :]
    b2d = lax.broadcast_in_dim(batch_ref[...], (N, NG), (0,))
    gi = lax.broadcasted_iota(jnp.int32, (N, NG), 1)
    onehot = (b2d == gi).astype(jnp.float32)
    sums = lax.dot_general(onehot, hn, (((0,), (0,)), ((), ())),
                           preferred_element_type=jnp.float32)
    cnt = jnp.sum(onehot, axis=0)[:, None]
    pooled = sums / jnp.maximum(cnt, 1.0)
    out_ref[...] = (jnp.dot(pooled, wl_ref[...],
                            preferred_element_type=jnp.float32)
                    + bl_ref[...])


def _tc_call(body, out_shapes, *args):
    return pl.pallas_call(
        body,
        out_shape=[jax.ShapeDtypeStruct(s, jnp.float32) for s in out_shapes],
    )(*args)


def kernel(x, edge_index, batch, atom_emb, W1, b1, g1, bt1, W2, b2, g2, bt2,
           Wl, bl):
    x = x.astype(jnp.int32)
    src = edge_index[0].astype(jnp.int32)
    dst = edge_index[1].astype(jnp.int32)
    batch = batch.astype(jnp.int32)

    emb = atom_emb.reshape(NF * atom_emb.shape[1], H)
    xidx = x + (jnp.arange(NF, dtype=jnp.int32) * atom_emb.shape[1])[None, :]
    h0 = jnp.zeros((N, H), jnp.float32)
    for f in range(NF):
        h0 = h0 + jnp.take(emb, xidx[:, f], axis=0)
    h0 = jnp.pad(h0, ((0, NPAD - N), (0, 0)))

    degv = jnp.zeros((NPAD,), jnp.float32).at[dst].add(1.0)
    deg2 = jnp.stack([degv, jnp.zeros((NPAD,), jnp.float32)], axis=1)

    def conv_sc(hwa, hwb):
        hw = jnp.concatenate([hwa, hwb], axis=1)
        agg = jnp.zeros((NPAD, H), jnp.float32).at[dst].add(
            jnp.take(hw, src, axis=0))
        return agg[:, :HH], agg[:, HH:]

    hw1a, hw1b = _tc_call(_tc1_body, [(NPAD, HH), (NPAD, HH)],
                          h0[:, :HH], h0[:, HH:], deg2, W1)
    agg1a, agg1b = conv_sc(hw1a, hw1b)
    hw2a, hw2b = _tc_call(_tc2_body, [(NPAD, HH), (NPAD, HH)],
                          agg1a, agg1b, hw1a, hw1b, deg2,
                          b1[None, :], g1[None, :], bt1[None, :], W2)
    agg2a, agg2b = conv_sc(hw2a, hw2b)
    (out,) = _tc_call(_tc3_body, [(NG, H)],
                      agg2a, agg2b, hw2a, hw2b, deg2,
                      b2[None, :], g2[None, :], bt2[None, :], batch,
                      Wl, bl[None, :])
    return out


# speedup vs baseline: 19.0510x; 7.7382x over previous
import numpy as np

import jax
import jax.numpy as jnp
from jax import lax
from jax.experimental import pallas as pl
from jax.experimental.pallas import tpu as pltpu
from jax.experimental.pallas import tpu_sc as plsc

N = 10000
H = 128
NF = 9
NG = 64
NC = 2
NS = 16
NW = NC * NS
NPAD = 10240
NCPC = 40
NPC = NCPC * 128
TPC = NPAD // NS
E2 = 327680
EPT = E2 // NW
KE = E2 // NW // 128
NBLK = 16


def _lchunk(sid):
    return jnp.where(sid < 8, 3 * sid, 8 + 2 * sid)


def _zero_tile_buf(buf):
    def body(r, c):
        for t in range(H // 16):
            buf[r, pl.ds(t * 16, 16)] = jnp.zeros((16,), jnp.float32)
        return c
    lax.fori_loop(0, 128, body, 0)


def _sc_deg_body(dste_hbm, degp_hbm, dstb_v, degl_v):
    cid = lax.axis_index("c")
    sid = lax.axis_index("s")
    w = cid * NS + sid

    def zdeg(i, c):
        degl_v[pl.ds(i * 16, 16)] = jnp.zeros((16,), jnp.float32)
        return c
    lax.fori_loop(0, NPAD // 16, zdeg, 0)

    pltpu.sync_copy(dste_hbm.at[pl.ds(w * EPT, EPT)], dstb_v)
    ones16 = jnp.ones((16,), jnp.float32)

    def dbody(j, c):
        idx = dstb_v[pl.ds(j * 16, 16)]
        plsc.addupdate_scatter(degl_v, [idx], ones16)
        return c
    lax.fori_loop(0, EPT // 16, dbody, 0)
    pltpu.sync_copy(degl_v, degp_hbm.at[pl.ds(w * NPAD, NPAD)])


def _sc_enc_body(emb_hbm, xidx_hbm, h0_hbm,
                 h0_sh, xall_v, lin_v, t0, t1,
                 g0, g1, s0, s1):
    cid = lax.axis_index("c")
    sid = lax.axis_index("s")
    w = cid * NS + sid
    qcnt = jnp.where(sid < 8, 3, 2)
    lbase = _lchunk(sid) * 128
    nit = NF * qcnt
    tmp = (t0, t1)
    gs = (g0, g1)
    ss = (s0, s1)

    for q in range(3):
        for t in range(8):
            lin_v[q, pl.ds(t * 16, 16)] = (
                lbase + q * 128 + t * 16 + lax.iota(jnp.int32, 16))
    _zero_tile_buf(t0)

    pltpu.sync_copy(xidx_hbm.at[w], xall_v)

    @pl.when(sid < 8)
    def _():
        for q in range(3):
            pltpu.sync_copy(t0, h0_sh.at[pl.ds(lbase + q * 128, 128)])

    @pl.when(sid >= 8)
    def _():
        for q in range(2):
            pltpu.sync_copy(t0, h0_sh.at[pl.ds(lbase + q * 128, 128)])
    plsc.subcore_barrier()

    def ebody(p, c):
        ks = [2 * p + u for u in range(2)]
        for u in range(2):
            @pl.when(ks[u] < nit)
            def _():
                @pl.when(p > 0)
                def _():
                    pltpu.make_async_copy(
                        tmp[u], h0_sh.at[pl.ds(0, 128)], ss[u]).wait()
                f = lax.div(ks[u], qcnt)
                qi = lax.rem(ks[u], qcnt)
                pltpu.async_copy(emb_hbm.at[xall_v.at[f, qi]], tmp[u], gs[u])
        for u in range(2):
            @pl.when(ks[u] < nit)
            def _():
                pltpu.make_async_copy(
                    emb_hbm.at[pl.ds(0, 128)], tmp[u], gs[u]).wait()
                qi = lax.rem(ks[u], qcnt)
                pltpu.async_copy(tmp[u], h0_sh.at[lin_v.at[qi]],
                                 ss[u], add=True)
        return c
    lax.fori_loop(0, (NF * 3 + 1) // 2, ebody, 0)
    for u in range(2):
        pltpu.make_async_copy(tmp[u], h0_sh.at[pl.ds(0, 128)], ss[u]).wait()
    plsc.subcore_barrier()

    @pl.when(sid < 8)
    def _():
        pltpu.sync_copy(h0_sh.at[pl.ds(lbase, 384)],
                        h0_hbm.at[pl.ds(cid * NPC + lbase, 384)])

    @pl.when(sid >= 8)
    def _():
        pltpu.sync_copy(h0_sh.at[pl.ds(lbase, 256)],
                        h0_hbm.at[pl.ds(cid * NPC + lbase, 256)])


def _sc_conv_body(hw_hbm, srce_hbm, dste_hbm, aggp_hbm,
                  agg_sh, srcb_v, dstb_v, t0, t1,
                  g0, g1, s0, s1):
    cid = lax.axis_index("c")
    sid = lax.axis_index("s")
    w = cid * NS + sid
    nbase = sid * TPC
    tmp = (t0, t1)
    gs = (g0, g1)
    ss = (s0, s1)

    _zero_tile_buf(t0)
    for q in range(TPC // 128):
        pltpu.sync_copy(t0, agg_sh.at[pl.ds(nbase + q * 128, 128)])
    plsc.subcore_barrier()

    def blk_body(blk, c):
        @pl.when(blk > 0)
        def _():
            for u in range(2):
                pltpu.make_async_copy(
                    tmp[u], agg_sh.at[pl.ds(0, 128)], ss[u]).wait()
        pltpu.sync_copy(srce_hbm.at[w, pl.ds(blk * NBLK, NBLK)], srcb_v)
        pltpu.sync_copy(dste_hbm.at[w, pl.ds(blk * NBLK, NBLK)], dstb_v)

        def pair_body(pi, c2):
            for u in range(2):
                @pl.when(pi > 0)
                def _():
                    pltpu.make_async_copy(
                        tmp[u], agg_sh.at[pl.ds(0, 128)], ss[u]).wait()
                pltpu.async_copy(hw_hbm.at[srcb_v.at[2 * pi + u]],
                                 tmp[u], gs[u])
            for u in range(2):
                pltpu.make_async_copy(
                    hw_hbm.at[pl.ds(0, 128)], tmp[u], gs[u]).wait()
                pltpu.async_copy(tmp[u], agg_sh.at[dstb_v.at[2 * pi + u]],
                                 ss[u], add=True)
            return c2
        lax.fori_loop(0, NBLK // 2, pair_body, 0)
        return c
    lax.fori_loop(0, KE // NBLK, blk_body, 0)
    for u in range(2):
        pltpu.make_async_copy(tmp[u], agg_sh.at[pl.ds(0, 128)], ss[u]).wait()
    plsc.subcore_barrier()

    pltpu.sync_copy(agg_sh.at[pl.ds(nbase, TPC)],
                    aggp_hbm.at[pl.ds(cid * NPAD + nbase, TPC)])


_MESH = plsc.VectorSubcoreMesh(core_axis_name="c", subcore_axis_name="s",
                               num_cores=NC, num_subcores=NS)

_DEG_SCRATCH = [
    pltpu.VMEM((EPT,), jnp.int32),
    pltpu.VMEM((NPAD,), jnp.float32),
]

_ENC_SCRATCH = [
    pltpu.VMEM_SHARED((NPC, H), jnp.float32),
    pltpu.VMEM((NF, 3, 128), jnp.int32),
    pltpu.VMEM((3, 128), jnp.int32),
    pltpu.VMEM((128, H), jnp.float32),
    pltpu.VMEM((128, H), jnp.float32),
] + [pltpu.SemaphoreType.DMA] * 4

_CONV_SCRATCH = [
    pltpu.VMEM_SHARED((NPAD, H), jnp.float32),
    pltpu.VMEM((NBLK, 128), jnp.int32),
    pltpu.VMEM((NBLK, 128), jnp.int32),
    pltpu.VMEM((128, H), jnp.float32),
    pltpu.VMEM((128, H), jnp.float32),
] + [pltpu.SemaphoreType.DMA] * 4

_deg_call = pl.kernel(
    _sc_deg_body,
    out_type=[jax.ShapeDtypeStruct((NW * NPAD,), jnp.float32)],
    mesh=_MESH, scratch_types=_DEG_SCRATCH,
    compiler_params=pltpu.CompilerParams(needs_layout_passes=False))

_enc_call = pl.kernel(
    _sc_enc_body,
    out_type=[jax.ShapeDtypeStruct((NPAD, H), jnp.float32)],
    mesh=_MESH, scratch_types=_ENC_SCRATCH)

_conv_call = pl.kernel(
    _sc_conv_body,
    out_type=[jax.ShapeDtypeStruct((2 * NPAD, H), jnp.float32)],
    mesh=_MESH, scratch_types=_CONV_SCRATCH)


def _mask_rows(nrows):
    ri = lax.broadcasted_iota(jnp.int32, (nrows, 1), 0)
    return (ri < N).astype(jnp.float32)


def _dinv(degc_ref):
    return lax.rsqrt(degc_ref[...] + 1.0)


def _tc1_body(h0_ref, degpt_ref, w1_ref, out_ref, degc_ref):
    deg = jnp.sum(degpt_ref[...], axis=1, keepdims=True)
    degc_ref[...] = deg
    out_ref[...] = jnp.dot(h0_ref[...], w1_ref[...],
                           preferred_element_type=jnp.float32) \
        * lax.rsqrt(deg + 1.0)


def _bn_relu(z, g_ref, bt_ref, zero_pad):
    m = _mask_rows(z.shape[0])
    mu = jnp.sum(z * m, axis=0, keepdims=True) / N
    d = (z - mu) * m
    var = jnp.sum(d * d, axis=0, keepdims=True) / N
    h = jnp.maximum((z - mu) * lax.rsqrt(var + 1e-5) * g_ref[...]
                    + bt_ref[...], 0.0)
    return h * m if zero_pad else h


def _tc2_body(aggp_ref, hw_ref, degc_ref, b_ref, g_ref, bt_ref, w2_ref,
              out_ref):
    dinv = _dinv(degc_ref)
    z = (aggp_ref[:NPAD, :] + aggp_ref[NPAD:, :] + hw_ref[...]) * dinv \
        + b_ref[...]
    h = _bn_relu(z, g_ref, bt_ref, True)
    out_ref[...] = jnp.dot(h, w2_ref[...],
                           preferred_element_type=jnp.float32) * dinv


def _tc3_body(aggp_ref, hw_ref, degc_ref, b_ref, g_ref, bt_ref, batchp_ref,
              wl_ref, bl_ref, out_ref):
    z = (aggp_ref[:NPAD, :] + aggp_ref[NPAD:, :] + hw_ref[...]) \
        * _dinv(degc_ref) + b_ref[...]
    h = _bn_relu(z, g_ref, bt_ref, False)
    gi = lax.broadcasted_iota(jnp.int32, (NPAD, NG), 1)
    onehot = (batchp_ref[...] == gi).astype(jnp.float32)
    sums = lax.dot_general(onehot, h, (((0,), (0,)), ((), ())),
                           preferred_element_type=jnp.float32)
    cnt = lax.dot_general(onehot, jnp.ones((NPAD, 1), jnp.float32),
                          (((0,), (0,)), ((), ())),
                          preferred_element_type=jnp.float32)
    pooled = sums / jnp.maximum(cnt, 1.0)
    out_ref[...] = (jnp.dot(pooled, wl_ref[...],
                            preferred_element_type=jnp.float32) + bl_ref[...])


def _tc_call(body, out_shapes, *args):
    return pl.pallas_call(
        body,
        out_shape=[jax.ShapeDtypeStruct(s, jnp.float32) for s in out_shapes],
    )(*args)


def _chunk_map():
    m = np.zeros((NW, 3), np.int32)
    for w in range(NW):
        c, s = divmod(w, NS)
        base = c * NCPC + (3 * s if s < 8 else 8 + 2 * s)
        qc = 3 if s < 8 else 2
        for q in range(3):
            m[w, q] = base + min(q, qc - 1)
    return m


def kernel(x, edge_index, batch, atom_emb, W1, b1, g1, bt1, W2, b2, g2, bt2,
           Wl, bl):
    x = x.astype(jnp.int32)
    src = edge_index[0].astype(jnp.int32)
    dst = edge_index[1].astype(jnp.int32)
    batch = batch.astype(jnp.int32)
    voc = atom_emb.shape[1]

    emb = atom_emb.reshape(NF * voc, H)

    xidx = (x + (jnp.arange(NF, dtype=jnp.int32) * voc)[None, :]).T
    xidx = jnp.pad(xidx, ((0, 0), (0, NPAD - N)))
    xidxT = xidx.reshape(NF, NPAD // 128, 128)[:, _chunk_map(), :]
    xidxT = xidxT.transpose(1, 0, 2, 3)

    padn = E2 - src.shape[0]
    ar = jnp.arange(padn, dtype=jnp.int32)
    src_p = jnp.concatenate([src, ar % N])
    dst_p = jnp.concatenate([dst, N + ar % (NPAD - N)])
    srce = src_p.reshape(NW, KE, 128)
    dste = dst_p.reshape(NW, KE, 128)

    batchp = jnp.pad(batch, (0, NPAD - N), constant_values=NG)[:, None]

    (degp,) = _deg_call(dst_p)
    (h0,) = _enc_call(emb, xidxT)
    degpT = degp.reshape(NW, NPAD).T

    (hw1, degc) = _tc_call(_tc1_body, [(NPAD, H), (NPAD, 1)],
                           h0, degpT, W1)
    (agg1p,) = _conv_call(hw1, srce, dste)
    (hw2,) = _tc_call(_tc2_body, [(NPAD, H)], agg1p, hw1, degc,
                      b1[None, :], g1[None, :], bt1[None, :], W2)
    (agg2p,) = _conv_call(hw2, srce, dste)
    (out,) = _tc_call(_tc3_body, [(NG, H)], agg2p, hw2, degc,
                      b2[None, :], g2[None, :], bt2[None, :],
                      batchp, Wl, bl[None, :])
    return out


# speedup vs baseline: 19.5832x; 1.0279x over previous
import numpy as np

import jax
import jax.numpy as jnp
from jax import lax
from jax.experimental import pallas as pl
from jax.experimental.pallas import tpu as pltpu
from jax.experimental.pallas import tpu_sc as plsc

N = 10000
H = 128
NF = 9
NG = 64
NC = 2
NS = 16
NW = NC * NS
NPAD = 10240
NCPC = 40
NPC = NCPC * 128
TPC = NPAD // NS
E2 = 327680
EPT = E2 // NW
KE = E2 // NW // 128
NBLK = 40


def _lchunk(sid):
    return jnp.where(sid < 8, 3 * sid, 8 + 2 * sid)


def _zero_tile_buf(buf):
    def body(r, c):
        for t in range(H // 16):
            buf[r, pl.ds(t * 16, 16)] = jnp.zeros((16,), jnp.float32)
        return c
    lax.fori_loop(0, 128, body, 0)


def _sc_deg_body(dste_hbm, degp_hbm, dstb_v, degl_v):
    cid = lax.axis_index("c")
    sid = lax.axis_index("s")
    w = cid * NS + sid

    def zdeg(i, c):
        degl_v[pl.ds(i * 16, 16)] = jnp.zeros((16,), jnp.float32)
        return c
    lax.fori_loop(0, NPAD // 16, zdeg, 0)

    pltpu.sync_copy(dste_hbm.at[pl.ds(w * EPT, EPT)], dstb_v)
    ones16 = jnp.ones((16,), jnp.float32)

    def dbody(j, c):
        idx = dstb_v[pl.ds(j * 16, 16)]
        plsc.addupdate_scatter(degl_v, [idx], ones16)
        return c
    lax.fori_loop(0, EPT // 16, dbody, 0)
    pltpu.sync_copy(degl_v, degp_hbm.at[pl.ds(w * NPAD, NPAD)])


def _sc_enc_body(emb_hbm, xidx_hbm, h0_hbm,
                 h0_sh, xall_v, lin_v, t0, t1, t2, t3,
                 g0, g1, g2, g3, s0, s1, s2, s3):
    cid = lax.axis_index("c")
    sid = lax.axis_index("s")
    w = cid * NS + sid
    qcnt = jnp.where(sid < 8, 3, 2)
    lbase = _lchunk(sid) * 128
    nit = NF * qcnt
    tmp = (t0, t1, t2, t3)
    gs = (g0, g1, g2, g3)
    ss = (s0, s1, s2, s3)

    for q in range(3):
        for t in range(8):
            lin_v[q, pl.ds(t * 16, 16)] = (
                lbase + q * 128 + t * 16 + lax.iota(jnp.int32, 16))
    _zero_tile_buf(t0)

    pltpu.sync_copy(xidx_hbm.at[w], xall_v)

    @pl.when(sid < 8)
    def _():
        for q in range(3):
            pltpu.sync_copy(t0, h0_sh.at[pl.ds(lbase + q * 128, 128)])

    @pl.when(sid >= 8)
    def _():
        for q in range(2):
            pltpu.sync_copy(t0, h0_sh.at[pl.ds(lbase + q * 128, 128)])
    plsc.subcore_barrier()

    def ebody(p, c):
        ks = [4 * p + b for b in range(4)]
        for b in range(4):
            @pl.when(ks[b] < nit)
            def _():
                @pl.when(p > 0)
                def _():
                    pltpu.make_async_copy(
                        tmp[b], h0_sh.at[pl.ds(0, 128)], ss[b]).wait()
                f = lax.div(ks[b], qcnt)
                qi = lax.rem(ks[b], qcnt)
                pltpu.async_copy(emb_hbm.at[xall_v.at[f, qi]], tmp[b], gs[b])
        for b in range(4):
            @pl.when(ks[b] < nit)
            def _():
                pltpu.make_async_copy(
                    emb_hbm.at[pl.ds(0, 128)], tmp[b], gs[b]).wait()
                qi = lax.rem(ks[b], qcnt)
                pltpu.async_copy(tmp[b], h0_sh.at[lin_v.at[qi]],
                                 ss[b], add=True)
        return c
    lax.fori_loop(0, (NF * 3 + 3) // 4, ebody, 0)
    for b in range(4):
        pltpu.make_async_copy(tmp[b], h0_sh.at[pl.ds(0, 128)], ss[b]).wait()
    plsc.subcore_barrier()

    @pl.when(sid < 8)
    def _():
        pltpu.sync_copy(h0_sh.at[pl.ds(lbase, 384)],
                        h0_hbm.at[pl.ds(cid * NPC + lbase, 384)])

    @pl.when(sid >= 8)
    def _():
        pltpu.sync_copy(h0_sh.at[pl.ds(lbase, 256)],
                        h0_hbm.at[pl.ds(cid * NPC + lbase, 256)])


def _sc_conv_body(hw_hbm, srce_hbm, dste_hbm, aggp_hbm,
                  agg_sh, srcb_v, dstb_v, t0, t1,
                  g0, g1, s0, s1):
    cid = lax.axis_index("c")
    sid = lax.axis_index("s")
    w = cid * NS + sid
    nbase = sid * TPC
    tmp = (t0, t1)
    gs = (g0, g1)
    ss = (s0, s1)

    _zero_tile_buf(t0)
    for q in range(TPC // 128):
        pltpu.sync_copy(t0, agg_sh.at[pl.ds(nbase + q * 128, 128)])
    plsc.subcore_barrier()

    def blk_body(blk, c):
        @pl.when(blk > 0)
        def _():
            for u in range(2):
                pltpu.make_async_copy(
                    tmp[u], agg_sh.at[pl.ds(0, 128)], ss[u]).wait()
        pltpu.sync_copy(srce_hbm.at[w, pl.ds(blk * NBLK, NBLK)], srcb_v)
        pltpu.sync_copy(dste_hbm.at[w, pl.ds(blk * NBLK, NBLK)], dstb_v)

        def pair_body(pi, c2):
            for u in range(2):
                @pl.when(pi > 0)
                def _():
                    pltpu.make_async_copy(
                        tmp[u], agg_sh.at[pl.ds(0, 128)], ss[u]).wait()
                pltpu.async_copy(hw_hbm.at[srcb_v.at[2 * pi + u]],
                                 tmp[u], gs[u])
            for u in range(2):
                pltpu.make_async_copy(
                    hw_hbm.at[pl.ds(0, 128)], tmp[u], gs[u]).wait()
                pltpu.async_copy(tmp[u], agg_sh.at[dstb_v.at[2 * pi + u]],
                                 ss[u], add=True)
            return c2
        lax.fori_loop(0, NBLK // 2, pair_body, 0)
        return c
    lax.fori_loop(0, KE // NBLK, blk_body, 0)
    for u in range(2):
        pltpu.make_async_copy(tmp[u], agg_sh.at[pl.ds(0, 128)], ss[u]).wait()
    plsc.subcore_barrier()

    pltpu.sync_copy(agg_sh.at[pl.ds(nbase, TPC)],
                    aggp_hbm.at[pl.ds(cid * NPAD + nbase, TPC)])


_MESH = plsc.VectorSubcoreMesh(core_axis_name="c", subcore_axis_name="s",
                               num_cores=NC, num_subcores=NS)

_DEG_SCRATCH = [
    pltpu.VMEM((EPT,), jnp.int32),
    pltpu.VMEM((NPAD,), jnp.float32),
]

_ENC_SCRATCH = [
    pltpu.VMEM_SHARED((NPC, H), jnp.float32),
    pltpu.VMEM((NF, 3, 128), jnp.int32),
    pltpu.VMEM((3, 128), jnp.int32),
    pltpu.VMEM((128, H), jnp.float32),
    pltpu.VMEM((128, H), jnp.float32),
    pltpu.VMEM((128, H), jnp.float32),
    pltpu.VMEM((128, H), jnp.float32),
] + [pltpu.SemaphoreType.DMA] * 8

_CONV_SCRATCH = [
    pltpu.VMEM_SHARED((NPAD, H), jnp.float32),
    pltpu.VMEM((NBLK, 128), jnp.int32),
    pltpu.VMEM((NBLK, 128), jnp.int32),
    pltpu.VMEM((128, H), jnp.float32),
    pltpu.VMEM((128, H), jnp.float32),
] + [pltpu.SemaphoreType.DMA] * 4

_deg_call = pl.kernel(
    _sc_deg_body,
    out_type=[jax.ShapeDtypeStruct((NW * NPAD,), jnp.float32)],
    mesh=_MESH, scratch_types=_DEG_SCRATCH,
    compiler_params=pltpu.CompilerParams(needs_layout_passes=False))

_enc_call = pl.kernel(
    _sc_enc_body,
    out_type=[jax.ShapeDtypeStruct((NPAD, H), jnp.float32)],
    mesh=_MESH, scratch_types=_ENC_SCRATCH)

_conv_call = pl.kernel(
    _sc_conv_body,
    out_type=[jax.ShapeDtypeStruct((2 * NPAD, H), jnp.float32)],
    mesh=_MESH, scratch_types=_CONV_SCRATCH)


def _mask_rows(nrows):
    ri = lax.broadcasted_iota(jnp.int32, (nrows, 1), 0)
    return (ri < N).astype(jnp.float32)


def _dinv(degc_ref):
    return lax.rsqrt(degc_ref[...] + 1.0)


def _tc1_body(h0_ref, degpt_ref, w1_ref, out_ref, degc_ref):
    deg = jnp.sum(degpt_ref[...], axis=1, keepdims=True)
    degc_ref[...] = deg
    out_ref[...] = jnp.dot(h0_ref[...], w1_ref[...],
                           preferred_element_type=jnp.float32) \
        * lax.rsqrt(deg + 1.0)


def _bn_relu(z, g_ref, bt_ref, zero_pad):
    m = _mask_rows(z.shape[0])
    mu = jnp.sum(z * m, axis=0, keepdims=True) / N
    d = (z - mu) * m
    var = jnp.sum(d * d, axis=0, keepdims=True) / N
    h = jnp.maximum((z - mu) * lax.rsqrt(var + 1e-5) * g_ref[...]
                    + bt_ref[...], 0.0)
    return h * m if zero_pad else h


def _combine(aggp_ref, hw_ref):
    a = aggp_ref[...].reshape(2, NPAD, H)
    return a[0] + a[1] + hw_ref[...]


def _tc2_body(aggp_ref, hw_ref, degc_ref, b_ref, g_ref, bt_ref,
              w2_ref, out_ref):
    dinv = _dinv(degc_ref)
    z = _combine(aggp_ref, hw_ref) * dinv + b_ref[...]
    h = _bn_relu(z, g_ref, bt_ref, True)
    out_ref[...] = jnp.dot(h, w2_ref[...],
                           preferred_element_type=jnp.float32) * dinv


def _tc3_body(aggp_ref, hw_ref, degc_ref, b_ref, g_ref, bt_ref,
              batchp_ref, wl_ref, bl_ref, out_ref):
    z = _combine(aggp_ref, hw_ref) * _dinv(degc_ref) + b_ref[...]
    h = _bn_relu(z, g_ref, bt_ref, False)
    gi = lax.broadcasted_iota(jnp.int32, (NPAD, NG), 1)
    onehot = (batchp_ref[...] == gi).astype(jnp.float32)
    sums = lax.dot_general(onehot, h, (((0,), (0,)), ((), ())),
                           preferred_element_type=jnp.float32)
    cnt = lax.dot_general(onehot, jnp.ones((NPAD, 1), jnp.float32),
                          (((0,), (0,)), ((), ())),
                          preferred_element_type=jnp.float32)
    pooled = sums / jnp.maximum(cnt, 1.0)
    out_ref[...] = (jnp.dot(pooled, wl_ref[...],
                            preferred_element_type=jnp.float32) + bl_ref[...])


def _tc_call(body, out_shapes, *args):
    return pl.pallas_call(
        body,
        out_shape=[jax.ShapeDtypeStruct(s, jnp.float32) for s in out_shapes],
    )(*args)


def _chunk_map():
    m = np.zeros((NW, 3), np.int32)
    for w in range(NW):
        c, s = divmod(w, NS)
        base = c * NCPC + (3 * s if s < 8 else 8 + 2 * s)
        qc = 3 if s < 8 else 2
        for q in range(3):
            m[w, q] = base + min(q, qc - 1)
    return m


def kernel(x, edge_index, batch, atom_emb, W1, b1, g1, bt1, W2, b2, g2, bt2,
           Wl, bl):
    x = x.astype(jnp.int32)
    src = edge_index[0].astype(jnp.int32)
    dst = edge_index[1].astype(jnp.int32)
    batch = batch.astype(jnp.int32)
    voc = atom_emb.shape[1]

    emb = atom_emb.reshape(NF * voc, H)

    xidx = (x + (jnp.arange(NF, dtype=jnp.int32) * voc)[None, :]).T
    xidx = jnp.pad(xidx, ((0, 0), (0, NPAD - N)))
    xidxT = xidx.reshape(NF, NPAD // 128, 128)[:, _chunk_map(), :]
    xidxT = xidxT.transpose(1, 0, 2, 3)

    padn = E2 - src.shape[0]
    ar = jnp.arange(padn, dtype=jnp.int32)
    src_p = jnp.concatenate([src, ar % N])
    dst_p = jnp.concatenate([dst, N + ar % (NPAD - N)])
    srce = src_p.reshape(NW, KE, 128)
    dste = dst_p.reshape(NW, KE, 128)

    batchp = jnp.pad(batch, (0, NPAD - N), constant_values=NG)[:, None]

    (degp,) = _deg_call(dst_p)
    (h0,) = _enc_call(emb, xidxT)
    degpT = degp.reshape(NW, NPAD).T

    (hw1, degc) = _tc_call(
        _tc1_body, [(NPAD, H), (NPAD, 1)], h0, degpT, W1)
    (agg1p,) = _conv_call(hw1, srce, dste)
    (hw2,) = _tc_call(
        _tc2_body, [(NPAD, H)], agg1p, hw1, degc,
        b1[None, :], g1[None, :], bt1[None, :], W2)
    (agg2p,) = _conv_call(hw2, srce, dste)
    (out,) = _tc_call(_tc3_body, [(NG, H)], agg2p, hw2, degc,
                      b2[None, :], g2[None, :], bt2[None, :],
                      batchp, Wl, bl[None, :])
    return out


# speedup vs baseline: 21.8177x; 1.1141x over previous
import numpy as np

import jax
import jax.numpy as jnp
from jax import lax
from jax.experimental import pallas as pl
from jax.experimental.pallas import tpu as pltpu
from jax.experimental.pallas import tpu_sc as plsc

N = 10000
H = 128
NF = 9
NG = 64
NC = 2
NS = 16
NW = NC * NS
NPAD = 10240
NCPC = 40
NPC = NCPC * 128
TPC = NPAD // NS
E2 = 327680
EPT = E2 // NW
KE = E2 // NW // 128
NBLK = 20


def _lchunk(sid):
    return jnp.where(sid < 8, 3 * sid, 8 + 2 * sid)


def _zero_tile_buf(buf):
    def body(r, c):
        for t in range(H // 16):
            buf[r, pl.ds(t * 16, 16)] = jnp.zeros((16,), jnp.float32)
        return c
    lax.fori_loop(0, buf.shape[0], body, 0)


def _sc_deg_body(dste_hbm, degp_hbm, dstb_v, degl_v):
    cid = lax.axis_index("c")
    sid = lax.axis_index("s")
    w = cid * NS + sid

    def zdeg(i, c):
        degl_v[pl.ds(i * 16, 16)] = jnp.zeros((16,), jnp.float32)
        return c
    lax.fori_loop(0, NPAD // 16, zdeg, 0)

    pltpu.sync_copy(dste_hbm.at[pl.ds(w * EPT, EPT)], dstb_v)
    ones16 = jnp.ones((16,), jnp.float32)

    def dbody(j, c):
        idx = dstb_v[pl.ds(j * 16, 16)]
        plsc.addupdate_scatter(degl_v, [idx], ones16)
        return c
    lax.fori_loop(0, EPT // 16, dbody, 0)
    pltpu.sync_copy(degl_v, degp_hbm.at[pl.ds(w * NPAD, NPAD)])


def _sc_enc_body(emb_hbm, xidx_hbm, h0_hbm,
                 h0_sh, xall_v, lin_v, t0, t1, t2, t3,
                 g0, g1, g2, g3, s0, s1, s2, s3):
    cid = lax.axis_index("c")
    sid = lax.axis_index("s")
    w = cid * NS + sid
    qcnt = jnp.where(sid < 8, 3, 2)
    lbase = _lchunk(sid) * 128
    nit = NF * qcnt
    tmp = (t0, t1, t2, t3)
    gs = (g0, g1, g2, g3)
    ss = (s0, s1, s2, s3)

    for q in range(3):
        for t in range(8):
            lin_v[q, pl.ds(t * 16, 16)] = (
                lbase + q * 128 + t * 16 + lax.iota(jnp.int32, 16))
    _zero_tile_buf(t0)

    pltpu.sync_copy(xidx_hbm.at[w], xall_v)

    @pl.when(sid < 8)
    def _():
        for q in range(3):
            pltpu.sync_copy(t0, h0_sh.at[pl.ds(lbase + q * 128, 128)])

    @pl.when(sid >= 8)
    def _():
        for q in range(2):
            pltpu.sync_copy(t0, h0_sh.at[pl.ds(lbase + q * 128, 128)])
    plsc.subcore_barrier()

    def ebody(p, c):
        ks = [4 * p + b for b in range(4)]
        for b in range(4):
            @pl.when(ks[b] < nit)
            def _():
                @pl.when(p > 0)
                def _():
                    pltpu.make_async_copy(
                        tmp[b], h0_sh.at[pl.ds(0, 128)], ss[b]).wait()
                f = lax.div(ks[b], qcnt)
                qi = lax.rem(ks[b], qcnt)
                pltpu.async_copy(emb_hbm.at[xall_v.at[f, qi]], tmp[b], gs[b])
        for b in range(4):
            @pl.when(ks[b] < nit)
            def _():
                pltpu.make_async_copy(
                    emb_hbm.at[pl.ds(0, 128)], tmp[b], gs[b]).wait()
                qi = lax.rem(ks[b], qcnt)
                pltpu.async_copy(tmp[b], h0_sh.at[lin_v.at[qi]],
                                 ss[b], add=True)
        return c
    lax.fori_loop(0, (NF * 3 + 3) // 4, ebody, 0)
    for b in range(4):
        pltpu.make_async_copy(tmp[b], h0_sh.at[pl.ds(0, 128)], ss[b]).wait()
    plsc.subcore_barrier()

    @pl.when(sid < 8)
    def _():
        pltpu.sync_copy(h0_sh.at[pl.ds(lbase, 384)],
                        h0_hbm.at[pl.ds(cid * NPC + lbase, 384)])

    @pl.when(sid >= 8)
    def _():
        pltpu.sync_copy(h0_sh.at[pl.ds(lbase, 256)],
                        h0_hbm.at[pl.ds(cid * NPC + lbase, 256)])


def _sc_conv_body(hw_hbm, srce_hbm, dste_hbm, aggp_hbm,
                  agg_sh, srcb_v, dstb_v, t0, t1, t2, t3,
                  g0, g1, g2, g3, s0, s1, s2, s3):
    cid = lax.axis_index("c")
    sid = lax.axis_index("s")
    w = cid * NS + sid
    nbase = sid * TPC
    tmp = (t0, t1, t2, t3)
    gs = (g0, g1, g2, g3)
    ss = (s0, s1, s2, s3)

    _zero_tile_buf(t0)
    for q in range(TPC // 64):
        pltpu.sync_copy(t0, agg_sh.at[pl.ds(nbase + q * 64, 64)])
    plsc.subcore_barrier()

    def blk_body(blk, c):
        @pl.when(blk > 0)
        def _():
            for u in range(4):
                pltpu.make_async_copy(
                    tmp[u], agg_sh.at[pl.ds(0, 64)], ss[u]).wait()
        pltpu.sync_copy(srce_hbm.at[w, pl.ds(blk * 2 * NBLK, 2 * NBLK)],
                        srcb_v)
        pltpu.sync_copy(dste_hbm.at[w, pl.ds(blk * 2 * NBLK, 2 * NBLK)],
                        dstb_v)

        def quad_body(pi, c2):
            for u in range(4):
                @pl.when(pi > 0)
                def _():
                    pltpu.make_async_copy(
                        tmp[u], agg_sh.at[pl.ds(0, 64)], ss[u]).wait()
                pltpu.async_copy(hw_hbm.at[srcb_v.at[4 * pi + u]],
                                 tmp[u], gs[u])
            for u in range(4):
                pltpu.make_async_copy(
                    hw_hbm.at[pl.ds(0, 64)], tmp[u], gs[u]).wait()
                pltpu.async_copy(tmp[u], agg_sh.at[dstb_v.at[4 * pi + u]],
                                 ss[u], add=True)
            return c2
        lax.fori_loop(0, 2 * NBLK // 4, quad_body, 0)
        return c
    lax.fori_loop(0, KE // NBLK, blk_body, 0)
    for u in range(4):
        pltpu.make_async_copy(tmp[u], agg_sh.at[pl.ds(0, 64)], ss[u]).wait()
    plsc.subcore_barrier()

    pltpu.sync_copy(agg_sh.at[pl.ds(nbase, TPC)],
                    aggp_hbm.at[pl.ds(cid * NPAD + nbase, TPC)])


_MESH = plsc.VectorSubcoreMesh(core_axis_name="c", subcore_axis_name="s",
                               num_cores=NC, num_subcores=NS)

_DEG_SCRATCH = [
    pltpu.VMEM((EPT,), jnp.int32),
    pltpu.VMEM((NPAD,), jnp.float32),
]

_ENC_SCRATCH = [
    pltpu.VMEM_SHARED((NPC, H), jnp.float32),
    pltpu.VMEM((NF, 3, 128), jnp.int32),
    pltpu.VMEM((3, 128), jnp.int32),
    pltpu.VMEM((128, H), jnp.float32),
    pltpu.VMEM((128, H), jnp.float32),
    pltpu.VMEM((128, H), jnp.float32),
    pltpu.VMEM((128, H), jnp.float32),
] + [pltpu.SemaphoreType.DMA] * 8

_CONV_SCRATCH = [
    pltpu.VMEM_SHARED((NPAD, H), jnp.float32),
    pltpu.VMEM((2 * NBLK, 64), jnp.int32),
    pltpu.VMEM((2 * NBLK, 64), jnp.int32),
    pltpu.VMEM((64, H), jnp.float32),
    pltpu.VMEM((64, H), jnp.float32),
    pltpu.VMEM((64, H), jnp.float32),
    pltpu.VMEM((64, H), jnp.float32),
] + [pltpu.SemaphoreType.DMA] * 8

_deg_call = pl.kernel(
    _sc_deg_body,
    out_type=[jax.ShapeDtypeStruct((NW * NPAD,), jnp.float32)],
    mesh=_MESH, scratch_types=_DEG_SCRATCH,
    compiler_params=pltpu.CompilerParams(needs_layout_passes=False))

_enc_call = pl.kernel(
    _sc_enc_body,
    out_type=[jax.ShapeDtypeStruct((NPAD, H), jnp.float32)],
    mesh=_MESH, scratch_types=_ENC_SCRATCH)

_conv_call = pl.kernel(
    _sc_conv_body,
    out_type=[jax.ShapeDtypeStruct((2 * NPAD, H), jnp.float32)],
    mesh=_MESH, scratch_types=_CONV_SCRATCH)


def _mask_rows(nrows):
    ri = lax.broadcasted_iota(jnp.int32, (nrows, 1), 0)
    return (ri < N).astype(jnp.float32)


def _dinv(degc_ref):
    return lax.rsqrt(degc_ref[...] + 1.0)


def _tc1_body(h0_ref, degpt_ref, w1_ref, out_ref, degc_ref):
    deg = jnp.sum(degpt_ref[...], axis=1, keepdims=True)
    degc_ref[...] = deg
    out_ref[...] = jnp.dot(h0_ref[...], w1_ref[...],
                           preferred_element_type=jnp.float32) \
        * lax.rsqrt(deg + 1.0)


def _bn_relu(z, g_ref, bt_ref, zero_pad):
    m = _mask_rows(z.shape[0])
    mu = jnp.sum(z * m, axis=0, keepdims=True) / N
    d = (z - mu) * m
    var = jnp.sum(d * d, axis=0, keepdims=True) / N
    h = jnp.maximum((z - mu) * lax.rsqrt(var + 1e-5) * g_ref[...]
                    + bt_ref[...], 0.0)
    return h * m if zero_pad else h


def _combine(aggp_ref, hw_ref):
    a = aggp_ref[...].reshape(2, NPAD, H)
    return a[0] + a[1] + hw_ref[...]


def _tc2_body(aggp_ref, hw_ref, degc_ref, b_ref, g_ref, bt_ref,
              w2_ref, out_ref):
    dinv = _dinv(degc_ref)
    z = _combine(aggp_ref, hw_ref) * dinv + b_ref[...]
    h = _bn_relu(z, g_ref, bt_ref, True)
    out_ref[...] = jnp.dot(h, w2_ref[...],
                           preferred_element_type=jnp.float32) * dinv


def _tc3_body(aggp_ref, hw_ref, degc_ref, b_ref, g_ref, bt_ref,
              batchp_ref, wl_ref, bl_ref, out_ref):
    z = _combine(aggp_ref, hw_ref) * _dinv(degc_ref) + b_ref[...]
    h = _bn_relu(z, g_ref, bt_ref, False)
    gi = lax.broadcasted_iota(jnp.int32, (NPAD, NG), 1)
    onehot = (batchp_ref[...] == gi).astype(jnp.float32)
    sums = lax.dot_general(onehot, h, (((0,), (0,)), ((), ())),
                           preferred_element_type=jnp.float32)
    cnt = lax.dot_general(onehot, jnp.ones((NPAD, 1), jnp.float32),
                          (((0,), (0,)), ((), ())),
                          preferred_element_type=jnp.float32)
    pooled = sums / jnp.maximum(cnt, 1.0)
    out_ref[...] = (jnp.dot(pooled, wl_ref[...],
                            preferred_element_type=jnp.float32) + bl_ref[...])


def _tc_call(body, out_shapes, *args):
    return pl.pallas_call(
        body,
        out_shape=[jax.ShapeDtypeStruct(s, jnp.float32) for s in out_shapes],
    )(*args)


def _chunk_map():
    m = np.zeros((NW, 3), np.int32)
    for w in range(NW):
        c, s = divmod(w, NS)
        base = c * NCPC + (3 * s if s < 8 else 8 + 2 * s)
        qc = 3 if s < 8 else 2
        for q in range(3):
            m[w, q] = base + min(q, qc - 1)
    return m


def kernel(x, edge_index, batch, atom_emb, W1, b1, g1, bt1, W2, b2, g2, bt2,
           Wl, bl):
    x = x.astype(jnp.int32)
    src = edge_index[0].astype(jnp.int32)
    dst = edge_index[1].astype(jnp.int32)
    batch = batch.astype(jnp.int32)
    voc = atom_emb.shape[1]

    emb = atom_emb.reshape(NF * voc, H)

    xidx = (x + (jnp.arange(NF, dtype=jnp.int32) * voc)[None, :]).T
    xidx = jnp.pad(xidx, ((0, 0), (0, NPAD - N)))
    xidxT = xidx.reshape(NF, NPAD // 128, 128)[:, _chunk_map(), :]
    xidxT = xidxT.transpose(1, 0, 2, 3)

    padn = E2 - src.shape[0]
    ar = jnp.arange(padn, dtype=jnp.int32)
    src_p = jnp.concatenate([src, ar % N])
    dst_p = jnp.concatenate([dst, N + ar % (NPAD - N)])
    srce = src_p.reshape(NW, 2 * KE, 64)
    dste = dst_p.reshape(NW, 2 * KE, 64)

    batchp = jnp.pad(batch, (0, NPAD - N), constant_values=NG)[:, None]

    (degp,) = _deg_call(dst_p)
    (h0,) = _enc_call(emb, xidxT)
    degpT = degp.reshape(NW, NPAD).T

    (hw1, degc) = _tc_call(
        _tc1_body, [(NPAD, H), (NPAD, 1)], h0, degpT, W1)
    (agg1p,) = _conv_call(hw1, srce, dste)
    (hw2,) = _tc_call(
        _tc2_body, [(NPAD, H)], agg1p, hw1, degc,
        b1[None, :], g1[None, :], bt1[None, :], W2)
    (agg2p,) = _conv_call(hw2, srce, dste)
    (out,) = _tc_call(_tc3_body, [(NG, H)], agg2p, hw2, degc,
                      b2[None, :], g2[None, :], bt2[None, :],
                      batchp, Wl, bl[None, :])
    return out


# speedup vs baseline: 21.9469x; 1.0059x over previous
import numpy as np

import jax
import jax.numpy as jnp
from jax import lax
from jax.experimental import pallas as pl
from jax.experimental.pallas import tpu as pltpu
from jax.experimental.pallas import tpu_sc as plsc

N = 10000
H = 128
NF = 9
NG = 64
NC = 2
NS = 16
NW = NC * NS
NPAD = 10240
NCPC = 40
NPC = NCPC * 128
TPC = NPAD // NS
E2 = 327680
EPT = E2 // NW
KE = E2 // NW // 128
NBLK = 20


def _lchunk(sid):
    return jnp.where(sid < 8, 3 * sid, 8 + 2 * sid)


def _zero_tile_buf(buf):
    def body(r, c):
        for t in range(H // 16):
            buf[r, pl.ds(t * 16, 16)] = jnp.zeros((16,), jnp.float32)
        return c
    lax.fori_loop(0, buf.shape[0], body, 0)


def _sc_enc_body(emb_hbm, gidx_hbm, tgt_hbm, dste_hbm, h0_hbm, degp_hbm,
                 h0_sh, gidxb, tgtb, t0, t1, t2, t3, dstb_v, degl_v,
                 g0, g1, g2, g3, s0, s1, s2, s3):
    cid = lax.axis_index("c")
    sid = lax.axis_index("s")
    w = cid * NS + sid
    nit = jnp.where(sid < 8, 23, 22)
    lbase = _lchunk(sid) * 128
    tmp = (t0, t1, t2, t3)
    gs = (g0, g1, g2, g3)
    ss = (s0, s1, s2, s3)

    pltpu.sync_copy(gidx_hbm.at[w], gidxb)
    pltpu.sync_copy(tgt_hbm.at[w], tgtb)
    _zero_tile_buf(t0)

    @pl.when(sid < 8)
    def _():
        for q in range(3):
            pltpu.sync_copy(t0, h0_sh.at[pl.ds(lbase + q * 128, 128)])

    @pl.when(sid >= 8)
    def _():
        for q in range(2):
            pltpu.sync_copy(t0, h0_sh.at[pl.ds(lbase + q * 128, 128)])
    plsc.subcore_barrier()

    def ebody(p, c):
        ks = [4 * p + b for b in range(4)]
        for b in range(4):
            @pl.when(ks[b] < nit)
            def _():
                @pl.when(p > 0)
                def _():
                    pltpu.make_async_copy(
                        tmp[b], h0_sh.at[pl.ds(0, 128)], ss[b]).wait()
                pltpu.async_copy(emb_hbm.at[gidxb.at[ks[b]]], tmp[b], gs[b])
        for b in range(4):
            @pl.when(ks[b] < nit)
            def _():
                pltpu.make_async_copy(
                    emb_hbm.at[pl.ds(0, 128)], tmp[b], gs[b]).wait()
                pltpu.async_copy(tmp[b], h0_sh.at[tgtb.at[ks[b]]],
                                 ss[b], add=True)
        return c
    lax.fori_loop(0, 6, ebody, 0)

    def zdeg(i, c):
        degl_v[pl.ds(i * 16, 16)] = jnp.zeros((16,), jnp.float32)
        return c
    lax.fori_loop(0, NPAD // 16, zdeg, 0)
    ones16 = jnp.ones((16,), jnp.float32)
    for hh in range(2):
        pltpu.sync_copy(dste_hbm.at[pl.ds(w * EPT + hh * (EPT // 2),
                                          EPT // 2)], dstb_v)

        def dbody(j, c):
            idx = dstb_v[pl.ds(j * 16, 16)]
            plsc.addupdate_scatter(degl_v, [idx], ones16)
            return c
        lax.fori_loop(0, EPT // 2 // 16, dbody, 0)
    pltpu.sync_copy(degl_v, degp_hbm.at[pl.ds(w * NPAD, NPAD)])

    for b in range(4):
        pltpu.make_async_copy(tmp[b], h0_sh.at[pl.ds(0, 128)], ss[b]).wait()
    plsc.subcore_barrier()

    @pl.when(sid < 8)
    def _():
        pltpu.sync_copy(h0_sh.at[pl.ds(lbase, 384)],
                        h0_hbm.at[pl.ds(cid * NPC + lbase, 384)])

    @pl.when(sid >= 8)
    def _():
        pltpu.sync_copy(h0_sh.at[pl.ds(lbase, 256)],
                        h0_hbm.at[pl.ds(cid * NPC + lbase, 256)])


def _sc_conv_body(hw_hbm, srce_hbm, dste_hbm, aggp_hbm,
                  agg_sh, srcb_v, dstb_v, t0, t1, t2, t3,
                  g0, g1, g2, g3, s0, s1, s2, s3):
    cid = lax.axis_index("c")
    sid = lax.axis_index("s")
    w = cid * NS + sid
    nbase = sid * TPC
    tmp = (t0, t1, t2, t3)
    gs = (g0, g1, g2, g3)
    ss = (s0, s1, s2, s3)

    _zero_tile_buf(t0)
    for q in range(TPC // 64):
        pltpu.sync_copy(t0, agg_sh.at[pl.ds(nbase + q * 64, 64)])
    plsc.subcore_barrier()

    def blk_body(blk, c):
        @pl.when(blk > 0)
        def _():
            for u in range(4):
                pltpu.make_async_copy(
                    tmp[u], agg_sh.at[pl.ds(0, 64)], ss[u]).wait()
        pltpu.sync_copy(srce_hbm.at[w, pl.ds(blk * 2 * NBLK, 2 * NBLK)],
                        srcb_v)
        pltpu.sync_copy(dste_hbm.at[w, pl.ds(blk * 2 * NBLK, 2 * NBLK)],
                        dstb_v)

        def quad_body(pi, c2):
            for u in range(4):
                @pl.when(pi > 0)
                def _():
                    pltpu.make_async_copy(
                        tmp[u], agg_sh.at[pl.ds(0, 64)], ss[u]).wait()
                pltpu.async_copy(hw_hbm.at[srcb_v.at[4 * pi + u]],
                                 tmp[u], gs[u])
            for u in range(4):
                pltpu.make_async_copy(
                    hw_hbm.at[pl.ds(0, 64)], tmp[u], gs[u]).wait()
                pltpu.async_copy(tmp[u], agg_sh.at[dstb_v.at[4 * pi + u]],
                                 ss[u], add=True)
            return c2
        lax.fori_loop(0, 2 * NBLK // 4, quad_body, 0)
        return c
    lax.fori_loop(0, KE // NBLK, blk_body, 0)
    for u in range(4):
        pltpu.make_async_copy(tmp[u], agg_sh.at[pl.ds(0, 64)], ss[u]).wait()
    plsc.subcore_barrier()

    pltpu.sync_copy(agg_sh.at[pl.ds(nbase, TPC)],
                    aggp_hbm.at[pl.ds(cid * NPAD + nbase, TPC)])


_MESH = plsc.VectorSubcoreMesh(core_axis_name="c", subcore_axis_name="s",
                               num_cores=NC, num_subcores=NS)

_ENC_SCRATCH = [
    pltpu.VMEM_SHARED((NPC, H), jnp.float32),
    pltpu.VMEM((23, 128), jnp.int32),
    pltpu.VMEM((23, 128), jnp.int32),
    pltpu.VMEM((128, H), jnp.float32),
    pltpu.VMEM((128, H), jnp.float32),
    pltpu.VMEM((128, H), jnp.float32),
    pltpu.VMEM((128, H), jnp.float32),
    pltpu.VMEM((EPT // 2,), jnp.int32),
    pltpu.VMEM((NPAD,), jnp.float32),
] + [pltpu.SemaphoreType.DMA] * 8

_CONV_SCRATCH = [
    pltpu.VMEM_SHARED((NPAD, H), jnp.float32),
    pltpu.VMEM((2 * NBLK, 64), jnp.int32),
    pltpu.VMEM((2 * NBLK, 64), jnp.int32),
    pltpu.VMEM((64, H), jnp.float32),
    pltpu.VMEM((64, H), jnp.float32),
    pltpu.VMEM((64, H), jnp.float32),
    pltpu.VMEM((64, H), jnp.float32),
] + [pltpu.SemaphoreType.DMA] * 8

_enc_call = pl.kernel(
    _sc_enc_body,
    out_type=[jax.ShapeDtypeStruct((NPAD, H), jnp.float32),
              jax.ShapeDtypeStruct((NW * NPAD,), jnp.float32)],
    mesh=_MESH, scratch_types=_ENC_SCRATCH,
    compiler_params=pltpu.CompilerParams(needs_layout_passes=False))

_conv_call = pl.kernel(
    _sc_conv_body,
    out_type=[jax.ShapeDtypeStruct((2 * NPAD, H), jnp.float32)],
    mesh=_MESH, scratch_types=_CONV_SCRATCH)


def _mask_rows(nrows):
    ri = lax.broadcasted_iota(jnp.int32, (nrows, 1), 0)
    return (ri < N).astype(jnp.float32)


def _dinv(degc_ref):
    return lax.rsqrt(degc_ref[...] + 1.0)


def _tc1_body(h0_ref, degpt_ref, w1_ref, out_ref, degc_ref):
    deg = jnp.sum(degpt_ref[...], axis=1, keepdims=True)
    degc_ref[...] = deg
    out_ref[...] = jnp.dot(h0_ref[...], w1_ref[...],
                           preferred_element_type=jnp.float32) \
        * lax.rsqrt(deg + 1.0)


def _bn_relu(z, g_ref, bt_ref, zero_pad):
    m = _mask_rows(z.shape[0])
    mu = jnp.sum(z * m, axis=0, keepdims=True) / N
    d = (z - mu) * m
    var = jnp.sum(d * d, axis=0, keepdims=True) / N
    h = jnp.maximum((z - mu) * lax.rsqrt(var + 1e-5) * g_ref[...]
                    + bt_ref[...], 0.0)
    return h * m if zero_pad else h


def _combine(aggp_ref, hw_ref):
    a = aggp_ref[...].reshape(2, NPAD, H)
    return a[0] + a[1] + hw_ref[...]


def _tc2_body(aggp_ref, hw_ref, degc_ref, b_ref, g_ref, bt_ref,
              w2_ref, out_ref):
    dinv = _dinv(degc_ref)
    z = _combine(aggp_ref, hw_ref) * dinv + b_ref[...]
    h = _bn_relu(z, g_ref, bt_ref, True)
    out_ref[...] = jnp.dot(h, w2_ref[...],
                           preferred_element_type=jnp.float32) * dinv


def _tc3_body(aggp_ref, hw_ref, degc_ref, b_ref, g_ref, bt_ref,
              batchp_ref, wl_ref, bl_ref, out_ref):
    z = _combine(aggp_ref, hw_ref) * _dinv(degc_ref) + b_ref[...]
    h = _bn_relu(z, g_ref, bt_ref, False)
    gi = lax.broadcasted_iota(jnp.int32, (NPAD, NG), 1)
    onehot = (batchp_ref[...] == gi).astype(jnp.float32)
    sums = lax.dot_general(onehot, h, (((0,), (0,)), ((), ())),
                           preferred_element_type=jnp.float32)
    cnt = lax.dot_general(onehot, jnp.ones((NPAD, 1), jnp.float32),
                          (((0,), (0,)), ((), ())),
                          preferred_element_type=jnp.float32)
    pooled = sums / jnp.maximum(cnt, 1.0)
    out_ref[...] = (jnp.dot(pooled, wl_ref[...],
                            preferred_element_type=jnp.float32) + bl_ref[...])


def _tc_call(body, out_shapes, *args):
    return pl.pallas_call(
        body,
        out_shape=[jax.ShapeDtypeStruct(s, jnp.float32) for s in out_shapes],
    )(*args)


def _item_maps():
    fm = np.zeros((NW, 23), np.int32)
    gm = np.zeros((NW, 23), np.int32)
    tgt = np.zeros((NW, 23, 128), np.int32)
    for w in range(NW):
        c, s = divmod(w, NS)
        lo = 23 * s if s < 8 else 184 + 22 * (s - 8)
        ns = 23 if s < 8 else 22
        for j in range(23):
            i = lo + min(j, ns - 1)
            f, q = divmod(i, NCPC)
            fm[w, j] = f
            gm[w, j] = c * NCPC + q
            tgt[w, j] = q * 128 + np.arange(128)
    return fm, gm, tgt


def kernel(x, edge_index, batch, atom_emb, W1, b1, g1, bt1, W2, b2, g2, bt2,
           Wl, bl):
    x = x.astype(jnp.int32)
    src = edge_index[0].astype(jnp.int32)
    dst = edge_index[1].astype(jnp.int32)
    batch = batch.astype(jnp.int32)
    voc = atom_emb.shape[1]

    emb = atom_emb.reshape(NF * voc, H)

    xidx = (x + (jnp.arange(NF, dtype=jnp.int32) * voc)[None, :]).T
    xidx = jnp.pad(xidx, ((0, 0), (0, NPAD - N)))
    fm, gm, tgtT = _item_maps()
    gidxT = xidx.reshape(NF, NPAD // 128, 128)[fm, gm]

    padn = E2 - src.shape[0]
    ar = jnp.arange(padn, dtype=jnp.int32)
    src_p = jnp.concatenate([src, ar % N])
    dst_p = jnp.concatenate([dst, N + ar % (NPAD - N)])
    srce = src_p.reshape(NW, 2 * KE, 64)
    dste = dst_p.reshape(NW, 2 * KE, 64)

    batchp = jnp.pad(batch, (0, NPAD - N), constant_values=NG)[:, None]

    (h0, degp) = _enc_call(emb, gidxT, jnp.asarray(tgtT), dst_p)
    degpT = degp.reshape(NW, NPAD).T

    (hw1, degc) = _tc_call(
        _tc1_body, [(NPAD, H), (NPAD, 1)], h0, degpT, W1)
    (agg1p,) = _conv_call(hw1, srce, dste)
    (hw2,) = _tc_call(
        _tc2_body, [(NPAD, H)], agg1p, hw1, degc,
        b1[None, :], g1[None, :], bt1[None, :], W2)
    (agg2p,) = _conv_call(hw2, srce, dste)
    (out,) = _tc_call(_tc3_body, [(NG, H)], agg2p, hw2, degc,
                      b2[None, :], g2[None, :], bt2[None, :],
                      batchp, Wl, bl[None, :])
    return out


# speedup vs baseline: 21.9837x; 1.0017x over previous
import numpy as np

import jax
import jax.numpy as jnp
from jax import lax
from jax.experimental import pallas as pl
from jax.experimental.pallas import tpu as pltpu
from jax.experimental.pallas import tpu_sc as plsc

N = 10000
H = 128
NF = 9
NG = 64
NC = 2
NS = 16
NW = NC * NS
NPAD = 10240
NCPC = 40
NPC = NCPC * 128
TPC = NPAD // NS
E2 = 327680
EPT = E2 // NW
KE = E2 // NW // 128
NBLK = 20


def _lchunk(sid):
    return jnp.where(sid < 8, 3 * sid, 8 + 2 * sid)


def _zero_tile_buf(buf):
    def body(r, c):
        for t in range(H // 16):
            buf[r, pl.ds(t * 16, 16)] = jnp.zeros((16,), jnp.float32)
        return c
    lax.fori_loop(0, buf.shape[0], body, 0)


def _sc_enc_body(emb_hbm, gidx_hbm, tgt_hbm, dste_hbm, h0_hbm, degp_hbm,
                 h0_sh, gidxb, tgtb, t0, t1, t2, t3, dstb_v, degl_v,
                 g0, g1, g2, g3, s0, s1, s2, s3):
    cid = lax.axis_index("c")
    sid = lax.axis_index("s")
    w = cid * NS + sid
    nit = jnp.where(sid < 8, 23, 22)
    lbase = _lchunk(sid) * 128
    tmp = (t0, t1, t2, t3)
    gs = (g0, g1, g2, g3)
    ss = (s0, s1, s2, s3)

    pltpu.sync_copy(gidx_hbm.at[w], gidxb)
    pltpu.sync_copy(tgt_hbm.at[w], tgtb)
    _zero_tile_buf(t0)

    for q in range(2):
        pltpu.async_copy(t0, h0_sh.at[pl.ds(lbase + q * 128, 128)], ss[q])

    @pl.when(sid < 8)
    def _():
        pltpu.async_copy(t0, h0_sh.at[pl.ds(lbase + 2 * 128, 128)], ss[2])
    for q in range(2):
        pltpu.make_async_copy(t0, h0_sh.at[pl.ds(0, 128)], ss[q]).wait()

    @pl.when(sid < 8)
    def _():
        pltpu.make_async_copy(t0, h0_sh.at[pl.ds(0, 128)], ss[2]).wait()
    plsc.subcore_barrier()

    def ebody(p, c):
        ks = [4 * p + b for b in range(4)]
        for b in range(4):
            @pl.when(ks[b] < nit)
            def _():
                @pl.when(p > 0)
                def _():
                    pltpu.make_async_copy(
                        tmp[b], h0_sh.at[pl.ds(0, 128)], ss[b]).wait()
                pltpu.async_copy(emb_hbm.at[gidxb.at[ks[b]]], tmp[b], gs[b])
        for b in range(4):
            @pl.when(ks[b] < nit)
            def _():
                pltpu.make_async_copy(
                    emb_hbm.at[pl.ds(0, 128)], tmp[b], gs[b]).wait()
                pltpu.async_copy(tmp[b], h0_sh.at[tgtb.at[ks[b]]],
                                 ss[b], add=True)
        return c
    lax.fori_loop(0, 6, ebody, 0)

    def zdeg(i, c):
        degl_v[pl.ds(i * 16, 16)] = jnp.zeros((16,), jnp.float32)
        return c
    lax.fori_loop(0, NPAD // 16, zdeg, 0)
    ones16 = jnp.ones((16,), jnp.float32)
    for hh in range(2):
        pltpu.sync_copy(dste_hbm.at[pl.ds(w * EPT + hh * (EPT // 2),
                                          EPT // 2)], dstb_v)

        def dbody(j, c):
            idx = dstb_v[pl.ds(j * 16, 16)]
            plsc.addupdate_scatter(degl_v, [idx], ones16)
            return c
        lax.fori_loop(0, EPT // 2 // 16, dbody, 0)
    pltpu.sync_copy(degl_v, degp_hbm.at[pl.ds(w * NPAD, NPAD)])

    for b in range(4):
        pltpu.make_async_copy(tmp[b], h0_sh.at[pl.ds(0, 128)], ss[b]).wait()
    plsc.subcore_barrier()

    @pl.when(sid < 8)
    def _():
        pltpu.sync_copy(h0_sh.at[pl.ds(lbase, 384)],
                        h0_hbm.at[pl.ds(cid * NPC + lbase, 384)])

    @pl.when(sid >= 8)
    def _():
        pltpu.sync_copy(h0_sh.at[pl.ds(lbase, 256)],
                        h0_hbm.at[pl.ds(cid * NPC + lbase, 256)])


def _sc_conv_body(hw_hbm, srce_hbm, dste_hbm, aggp_hbm,
                  agg_sh, srcb_v, dstb_v, t0, t1, t2, t3,
                  g0, g1, g2, g3, s0, s1, s2, s3):
    cid = lax.axis_index("c")
    sid = lax.axis_index("s")
    w = cid * NS + sid
    nbase = sid * TPC
    tmp = (t0, t1, t2, t3)
    gs = (g0, g1, g2, g3)
    ss = (s0, s1, s2, s3)

    _zero_tile_buf(t0)
    for q in range(TPC // 64):
        pltpu.async_copy(t0, agg_sh.at[pl.ds(nbase + q * 64, 64)],
                         ss[q % 4])
    for q in range(TPC // 64):
        pltpu.make_async_copy(t0, agg_sh.at[pl.ds(0, 64)],
                              ss[q % 4]).wait()
    plsc.subcore_barrier()

    def blk_body(blk, c):
        @pl.when(blk > 0)
        def _():
            for u in range(4):
                pltpu.make_async_copy(
                    tmp[u], agg_sh.at[pl.ds(0, 64)], ss[u]).wait()
        pltpu.sync_copy(srce_hbm.at[w, pl.ds(blk * 2 * NBLK, 2 * NBLK)],
                        srcb_v)
        pltpu.sync_copy(dste_hbm.at[w, pl.ds(blk * 2 * NBLK, 2 * NBLK)],
                        dstb_v)

        def quad_body(pi, c2):
            for u in range(4):
                @pl.when(pi > 0)
                def _():
                    pltpu.make_async_copy(
                        tmp[u], agg_sh.at[pl.ds(0, 64)], ss[u]).wait()
                pltpu.async_copy(hw_hbm.at[srcb_v.at[4 * pi + u]],
                                 tmp[u], gs[u])
            for u in range(4):
                pltpu.make_async_copy(
                    hw_hbm.at[pl.ds(0, 64)], tmp[u], gs[u]).wait()
                pltpu.async_copy(tmp[u], agg_sh.at[dstb_v.at[4 * pi + u]],
                                 ss[u], add=True)
            return c2
        lax.fori_loop(0, 2 * NBLK // 4, quad_body, 0)
        return c
    lax.fori_loop(0, KE // NBLK, blk_body, 0)
    for u in range(4):
        pltpu.make_async_copy(tmp[u], agg_sh.at[pl.ds(0, 64)], ss[u]).wait()
    plsc.subcore_barrier()

    pltpu.sync_copy(agg_sh.at[pl.ds(nbase, TPC)],
                    aggp_hbm.at[pl.ds(cid * NPAD + nbase, TPC)])


_MESH = plsc.VectorSubcoreMesh(core_axis_name="c", subcore_axis_name="s",
                               num_cores=NC, num_subcores=NS)

_ENC_SCRATCH = [
    pltpu.VMEM_SHARED((NPC, H), jnp.float32),
    pltpu.VMEM((23, 128), jnp.int32),
    pltpu.VMEM((23, 128), jnp.int32),
    pltpu.VMEM((128, H), jnp.float32),
    pltpu.VMEM((128, H), jnp.float32),
    pltpu.VMEM((128, H), jnp.float32),
    pltpu.VMEM((128, H), jnp.float32),
    pltpu.VMEM((EPT // 2,), jnp.int32),
    pltpu.VMEM((NPAD,), jnp.float32),
] + [pltpu.SemaphoreType.DMA] * 8

_CONV_SCRATCH = [
    pltpu.VMEM_SHARED((NPAD, H), jnp.float32),
    pltpu.VMEM((2 * NBLK, 64), jnp.int32),
    pltpu.VMEM((2 * NBLK, 64), jnp.int32),
    pltpu.VMEM((64, H), jnp.float32),
    pltpu.VMEM((64, H), jnp.float32),
    pltpu.VMEM((64, H), jnp.float32),
    pltpu.VMEM((64, H), jnp.float32),
] + [pltpu.SemaphoreType.DMA] * 8

_enc_call = pl.kernel(
    _sc_enc_body,
    out_type=[jax.ShapeDtypeStruct((NPAD, H), jnp.float32),
              jax.ShapeDtypeStruct((NW * NPAD,), jnp.float32)],
    mesh=_MESH, scratch_types=_ENC_SCRATCH,
    compiler_params=pltpu.CompilerParams(needs_layout_passes=False))

_conv_call = pl.kernel(
    _sc_conv_body,
    out_type=[jax.ShapeDtypeStruct((2 * NPAD, H), jnp.float32)],
    mesh=_MESH, scratch_types=_CONV_SCRATCH)


def _mask_rows(nrows):
    ri = lax.broadcasted_iota(jnp.int32, (nrows, 1), 0)
    return (ri < N).astype(jnp.float32)


def _dinv(degc_ref):
    return lax.rsqrt(degc_ref[...] + 1.0)


def _tc1_body(h0_ref, degpt_ref, w1_ref, out_ref, degc_ref):
    deg = jnp.sum(degpt_ref[...], axis=1, keepdims=True)
    degc_ref[...] = deg
    out_ref[...] = jnp.dot(h0_ref[...], w1_ref[...],
                           preferred_element_type=jnp.float32) \
        * lax.rsqrt(deg + 1.0)


def _bn_relu(z, g_ref, bt_ref, zero_pad):
    m = _mask_rows(z.shape[0])
    mu = jnp.sum(z * m, axis=0, keepdims=True) / N
    d = (z - mu) * m
    var = jnp.sum(d * d, axis=0, keepdims=True) / N
    h = jnp.maximum((z - mu) * lax.rsqrt(var + 1e-5) * g_ref[...]
                    + bt_ref[...], 0.0)
    return h * m if zero_pad else h


def _combine(aggp_ref, hw_ref):
    a = aggp_ref[...].reshape(2, NPAD, H)
    return a[0] + a[1] + hw_ref[...]


def _tc2_body(aggp_ref, hw_ref, degc_ref, b_ref, g_ref, bt_ref,
              w2_ref, out_ref):
    dinv = _dinv(degc_ref)
    z = _combine(aggp_ref, hw_ref) * dinv + b_ref[...]
    h = _bn_relu(z, g_ref, bt_ref, True)
    out_ref[...] = jnp.dot(h, w2_ref[...],
                           preferred_element_type=jnp.float32) * dinv


def _tc3_body(aggp_ref, hw_ref, degc_ref, b_ref, g_ref, bt_ref,
              batchp_ref, wl_ref, bl_ref, out_ref):
    z = _combine(aggp_ref, hw_ref) * _dinv(degc_ref) + b_ref[...]
    h = _bn_relu(z, g_ref, bt_ref, False)
    gi = lax.broadcasted_iota(jnp.int32, (NPAD, NG), 1)
    onehot = (batchp_ref[...] == gi).astype(jnp.float32)
    sums = lax.dot_general(onehot, h, (((0,), (0,)), ((), ())),
                           preferred_element_type=jnp.float32)
    cnt = lax.dot_general(onehot, jnp.ones((NPAD, 1), jnp.float32),
                          (((0,), (0,)), ((), ())),
                          preferred_element_type=jnp.float32)
    pooled = sums / jnp.maximum(cnt, 1.0)
    out_ref[...] = (jnp.dot(pooled, wl_ref[...],
                            preferred_element_type=jnp.float32) + bl_ref[...])


def _tc_call(body, out_shapes, *args):
    return pl.pallas_call(
        body,
        out_shape=[jax.ShapeDtypeStruct(s, jnp.float32) for s in out_shapes],
    )(*args)


def _item_maps():
    fm = np.zeros((NW, 23), np.int32)
    gm = np.zeros((NW, 23), np.int32)
    tgt = np.zeros((NW, 23, 128), np.int32)
    for w in range(NW):
        c, s = divmod(w, NS)
        lo = 23 * s if s < 8 else 184 + 22 * (s - 8)
        ns = 23 if s < 8 else 22
        for j in range(23):
            i = lo + min(j, ns - 1)
            f, q = divmod(i, NCPC)
            fm[w, j] = f
            gm[w, j] = c * NCPC + q
            tgt[w, j] = q * 128 + np.arange(128)
    return fm, gm, tgt


def kernel(x, edge_index, batch, atom_emb, W1, b1, g1, bt1, W2, b2, g2, bt2,
           Wl, bl):
    x = x.astype(jnp.int32)
    src = edge_index[0].astype(jnp.int32)
    dst = edge_index[1].astype(jnp.int32)
    batch = batch.astype(jnp.int32)
    voc = atom_emb.shape[1]

    emb = atom_emb.reshape(NF * voc, H)

    xidx = (x + (jnp.arange(NF, dtype=jnp.int32) * voc)[None, :]).T
    xidx = jnp.pad(xidx, ((0, 0), (0, NPAD - N)))
    fm, gm, tgtT = _item_maps()
    gidxT = xidx.reshape(NF, NPAD // 128, 128)[fm, gm]

    padn = E2 - src.shape[0]
    ar = jnp.arange(padn, dtype=jnp.int32)
    src_p = jnp.concatenate([src, ar % N])
    dst_p = jnp.concatenate([dst, N + ar % (NPAD - N)])
    srce = src_p.reshape(NW, 2 * KE, 64)
    dste = dst_p.reshape(NW, 2 * KE, 64)

    batchp = jnp.pad(batch, (0, NPAD - N), constant_values=NG)[:, None]

    (h0, degp) = _enc_call(emb, gidxT, jnp.asarray(tgtT), dst_p)
    degpT = degp.reshape(NW, NPAD).T

    (hw1, degc) = _tc_call(
        _tc1_body, [(NPAD, H), (NPAD, 1)], h0, degpT, W1)
    (agg1p,) = _conv_call(hw1, srce, dste)
    (hw2,) = _tc_call(
        _tc2_body, [(NPAD, H)], agg1p, hw1, degc,
        b1[None, :], g1[None, :], bt1[None, :], W2)
    (agg2p,) = _conv_call(hw2, srce, dste)
    (out,) = _tc_call(_tc3_body, [(NG, H)], agg2p, hw2, degc,
                      b2[None, :], g2[None, :], bt2[None, :],
                      batchp, Wl, bl[None, :])
    return out
